# Initial kernel scaffold; baseline (speedup 1.0000x reference)
import jax
import jax.numpy as jnp
from jax.experimental import pallas as pl


def kernel(x_seq, edge_index, edge_weight, h_enc, h_dec, params):
    raise NotImplementedError("write your pallas kernel here")


# speedup vs baseline: 1.0186x; 1.0186x over previous
import functools

import jax
import jax.numpy as jnp
from jax.experimental import pallas as pl

_N = 50000
_T = 4
_F_IN = 4
_H = 64
_LATENT = 32

_ROWS_BLK = 2000


def _final_linear(h, W, b):
    def body(h_ref, w_ref, b_ref, o_ref):
        o_ref[...] = jnp.maximum(h_ref[...], 0.0) @ w_ref[...] + b_ref[...]

    grid = (_N // _ROWS_BLK,)
    return pl.pallas_call(
        body,
        grid=grid,
        in_specs=[
            pl.BlockSpec((_ROWS_BLK, h.shape[1]), lambda i: (i, 0)),
            pl.BlockSpec(W.shape, lambda i: (0, 0)),
            pl.BlockSpec((1, b.shape[0]), lambda i: (0, 0)),
        ],
        out_specs=pl.BlockSpec((_ROWS_BLK, W.shape[1]), lambda i: (i, 0)),
        out_shape=jax.ShapeDtypeStruct((_N, W.shape[1]), jnp.float32),
    )(h, W, b.reshape(1, -1))


def _prop(xa, row, col, lhat):
    return jnp.zeros((_N, xa.shape[1]), jnp.float32).at[col].add(
        lhat[:, None] * xa[row])


def _gru_step(x, h, row, col, lhat, p, pre):
    W0x = jnp.concatenate([p[pre + 'xz_W0'], p[pre + 'xr_W0'], p[pre + 'xh_W0']], axis=1)
    W1x = jnp.concatenate([p[pre + 'xz_W1'], p[pre + 'xr_W1'], p[pre + 'xh_W1']], axis=1)
    bx = jnp.concatenate([p[pre + 'xz_b'], p[pre + 'xr_b'], p[pre + 'xh_b']])
    W0hzr = jnp.concatenate([p[pre + 'hz_W0'], p[pre + 'hr_W0']], axis=1)
    W1hzr = jnp.concatenate([p[pre + 'hz_W1'], p[pre + 'hr_W1']], axis=1)
    bhzr = jnp.concatenate([p[pre + 'hz_b'], p[pre + 'hr_b']])

    Px = _prop(x, row, col, lhat)
    Ph = _prop(h, row, col, lhat)
    Gx = x @ W0x + Px @ W1x + bx
    Gh = h @ W0hzr + Ph @ W1hzr + bhzr
    Z = jax.nn.sigmoid(Gx[:, :_H] + Gh[:, :_H])
    R = jax.nn.sigmoid(Gx[:, _H:2 * _H] + Gh[:, _H:])
    HR = h * R
    S = Gx[:, 2 * _H:] + HR @ p[pre + 'hh_W0'] + p[pre + 'hh_b']
    Phr = _prop(HR, row, col, lhat)
    Ht = jnp.tanh(S + Phr @ p[pre + 'hh_W1'])
    return Z * h + (1.0 - Z) * Ht


def kernel(x_seq, edge_index, edge_weight, h_enc, h_dec, params):
    row, col = edge_index[0], edge_index[1]
    deg = jnp.zeros((_N,), jnp.float32).at[row].add(edge_weight)
    dis = jnp.where(deg > 0, jax.lax.rsqrt(jnp.where(deg > 0, deg, 1.0)), 0.0)
    lhat = -(dis[row] * edge_weight * dis[col])

    y = None
    for t in range(_T):
        x = x_seq[t]
        h_enc = _gru_step(x, h_enc, row, col, lhat, params, 'enc_')
        z = jnp.maximum(h_enc, 0.0) @ params['enc_lin_W'] + params['enc_lin_b']
        h_dec = _gru_step(z, h_dec, row, col, lhat, params, 'dec_')
        y = _final_linear(h_dec, params['dec_lin_W'], params['dec_lin_b'])
    return y


# speedup vs baseline: 1.5341x; 1.5061x over previous
import functools

import jax
import jax.numpy as jnp
from jax import lax
from jax.experimental import pallas as pl
from jax.experimental.pallas import tpu as pltpu
from jax.experimental.pallas import tpu_sc as plsc

_N = 50000
_E = 1600000
_T = 4
_H = 64

_NC = 2
_NS = 16
_NW = _NC * _NS
_EPW = _E // _NW

_NPART = 64
_PR = 784
_NPAD = _NPART * _PR

_CH = 512
_CHSH = 9
_EPAD = _E + 2 * _CH
_CH2 = 2000

_BLK = 2000

_mesh = plsc.VectorSubcoreMesh(core_axis_name="c", subcore_axis_name="s")
_SC_PARAMS = pltpu.CompilerParams(use_tc_tiling_on_sc=False,
                                  needs_layout_passes=False)


@functools.partial(
    pl.kernel, mesh=_mesh, compiler_params=_SC_PARAMS,
    out_type=jax.ShapeDtypeStruct((_NC, _NPAD), jnp.float32),
    scratch_types=[
        pltpu.VMEM((_CH2,), jnp.int32),
        pltpu.VMEM((_CH2,), jnp.float32),
        pltpu.VMEM((_NPAD // _NS,), jnp.float32),
        pltpu.VMEM_SHARED((_NPAD,), jnp.float32),
    ],
)
def _deg_kernel(rows_hbm, w_hbm, out, ibuf, wbuf, zbuf, shacc):
    cid = lax.axis_index("c")
    sid = lax.axis_index("s")
    zero = jnp.zeros((16,), jnp.float32)
    zn = _NPAD // _NS

    def zb(j, c):
        zbuf[pl.ds(j * 16, 16)] = zero
        return c
    lax.fori_loop(0, zn // 16, zb, 0)
    pltpu.sync_copy(zbuf, shacc.at[pl.ds(pl.multiple_of(sid * zn, 8), zn)])
    plsc.subcore_barrier()

    wid = sid * _NC + cid
    base = wid * _EPW
    for k in range(_EPW // _CH2):
        s = base + k * _CH2
        pltpu.sync_copy(rows_hbm.at[pl.ds(s, _CH2)], ibuf)
        pltpu.sync_copy(w_hbm.at[pl.ds(s, _CH2)], wbuf)
        pltpu.sync_copy(wbuf, shacc.at[ibuf], add=True)
    plsc.subcore_barrier()

    @pl.when(sid == 0)
    def _():
        pltpu.sync_copy(shacc, out.at[cid])


@functools.partial(
    pl.kernel, mesh=_mesh, compiler_params=_SC_PARAMS,
    out_type=jax.ShapeDtypeStruct((_EPAD,), jnp.float32),
    scratch_types=[
        pltpu.VMEM((_CH2,), jnp.int32),
        pltpu.VMEM((_CH2,), jnp.int32),
        pltpu.VMEM((_CH2,), jnp.int32),
        pltpu.VMEM((_CH2,), jnp.float32),
        pltpu.VMEM((_CH2,), jnp.float32),
        pltpu.VMEM((_CH2,), jnp.float32),
        pltpu.VMEM((_CH2,), jnp.float32),
        pltpu.SemaphoreType.DMA,
    ],
)
def _lhat_kernel(perm_hbm, rows_hbm, cols_hbm, ew_hbm, dis_hbm, out,
                 pbuf, rbuf, cbuf, wbuf, drbuf, dcbuf, obuf, sem):
    cid = lax.axis_index("c")
    sid = lax.axis_index("s")
    wid = sid * _NC + cid
    base = wid * _EPW
    for k in range(_EPW // _CH2):
        s = base + k * _CH2
        pltpu.sync_copy(perm_hbm.at[pl.ds(s, _CH2)], pbuf)
        pltpu.sync_copy(rows_hbm.at[pl.ds(s, _CH2)], rbuf)
        pltpu.sync_copy(cols_hbm.at[pl.ds(s, _CH2)], cbuf)
        pltpu.async_copy(ew_hbm.at[pbuf], wbuf, sem).wait()
        pltpu.async_copy(dis_hbm.at[rbuf], drbuf, sem).wait()
        pltpu.async_copy(dis_hbm.at[cbuf], dcbuf, sem).wait()

        def vb(j, c):
            sl = pl.ds(j * 16, 16)
            obuf[sl] = -(drbuf[sl] * wbuf[sl] * dcbuf[sl])
            return c
        lax.fori_loop(0, _CH2 // 16, vb, 0)
        pltpu.sync_copy(obuf, out.at[pl.ds(s, _CH2)])


def _vextract(vec_ref, p):
    base = pl.multiple_of((p >> 4) << 4, 16)
    v = vec_ref[pl.ds(base, 16)]
    iota = lax.broadcasted_iota(jnp.int32, (16,), 0)
    return jnp.sum(jnp.where(iota == (p & 15), v, 0))


@functools.cache
def _make_prop(D):
    @functools.partial(
        pl.kernel, mesh=_mesh, compiler_params=_SC_PARAMS,
        out_type=jax.ShapeDtypeStruct((_NPAD, D), jnp.float32),
        scratch_types=[
            pltpu.VMEM((_PR, D), jnp.float32),
            pltpu.VMEM((_CH,), jnp.int32),
            pltpu.VMEM((_CH,), jnp.int32),
            pltpu.VMEM((_CH,), jnp.float32),
            pltpu.VMEM((80,), jnp.int32),
            pltpu.VMEM((_CH, D), jnp.float32),
            pltpu.SemaphoreType.DMA,
        ],
    )
    def _prop_kernel(table, rows_hbm, crel_hbm, lhat_hbm, eo_hbm, out,
                     acc, ibuf, cbuf, lbuf, eov, gbuf, sem):
        cid = lax.axis_index("c")
        sid = lax.axis_index("s")
        wid = sid * _NC + cid
        pltpu.sync_copy(eo_hbm, eov)
        zero = jnp.zeros((16,), jnp.float32)
        iota = lax.broadcasted_iota(jnp.int32, (16,), 0)
        for sub in range(2):
            p = wid * 2 + sub

            def zb(j, c):
                for r in range(D // 16):
                    acc[j, pl.ds(r * 16, 16)] = zero
                return c
            lax.fori_loop(0, _PR, zb, 0)

            e0 = _vextract(eov, p)
            e1 = _vextract(eov, p + 1)
            ab = (e0 >> 3) << 3
            nch = (e1 - ab + _CH - 1) >> _CHSH

            def chunk(k, c):
                ck = pl.multiple_of(ab + k * _CH, 8)
                pltpu.sync_copy(rows_hbm.at[pl.ds(ck, _CH)], ibuf)
                pltpu.sync_copy(crel_hbm.at[pl.ds(ck, _CH)], cbuf)
                pltpu.sync_copy(lhat_hbm.at[pl.ds(ck, _CH)], lbuf)
                pltpu.async_copy(table.at[ibuf], gbuf, sem).wait()
                lo = e0 - ck
                hi = e1 - ck

                def eb(g, cc):
                    b = g * 16
                    lane = b + iota
                    m = (lane >= lo) & (lane < hi)
                    crv = cbuf[pl.ds(b, 16)]
                    wv = lbuf[pl.ds(b, 16)]
                    rowi = lane
                    for l in range(D):
                        li = jnp.full((16,), l, jnp.int32)
                        colv = plsc.load_gather(gbuf, [rowi, li])
                        plsc.addupdate_scatter(acc, [crv, li], colv * wv,
                                               mask=m)
                    return cc
                lax.fori_loop(0, _CH // 16, eb, 0)
                return c
            lax.fori_loop(0, nch, chunk, 0)
            pltpu.sync_copy(acc, out.at[pl.ds(pl.multiple_of(p * _PR, 8), _PR)])

    return _prop_kernel


def _dis_tc(deg2):
    def body(a_ref, o_ref):
        d = a_ref[0] + a_ref[1]
        o_ref[...] = jnp.where(d > 0, lax.rsqrt(jnp.where(d > 0, d, 1.0)), 0.0)

    out = pl.pallas_call(
        body,
        out_shape=jax.ShapeDtypeStruct((_NPAD // 128, 128), jnp.float32),
    )(deg2.reshape(2, _NPAD // 128, 128))
    return out.reshape(_NPAD)


def _gates_tc(X, Px, H, Ph, W0x, W1x, bx, W0h, W1h, bh, W0hh, bhh):
    def body(x_ref, px_ref, h_ref, ph_ref, w0x_ref, w1x_ref, bx_ref,
             w0h_ref, w1h_ref, bh_ref, w0hh_ref, bhh_ref,
             z_ref, hr_ref, s_ref):
        gx = (x_ref[...] @ w0x_ref[...] + px_ref[...] @ w1x_ref[...]
              + bx_ref[...])
        gh = (h_ref[...] @ w0h_ref[...] + ph_ref[...] @ w1h_ref[...]
              + bh_ref[...])
        Z = jax.nn.sigmoid(gx[:, :_H] + gh[:, :_H])
        R = jax.nn.sigmoid(gx[:, _H:2 * _H] + gh[:, _H:])
        HR = h_ref[...] * R
        z_ref[...] = Z
        hr_ref[...] = HR
        s_ref[...] = gx[:, 2 * _H:] + HR @ w0hh_ref[...] + bhh_ref[...]

    dx = X.shape[1]
    grid = (_N // _BLK,)
    rows = lambda d: pl.BlockSpec((_BLK, d), lambda i: (i, 0))
    whole = lambda a: pl.BlockSpec(a.shape, lambda i: (0,) * a.ndim)
    o64 = jax.ShapeDtypeStruct((_N, _H), jnp.float32)
    return pl.pallas_call(
        body,
        grid=grid,
        in_specs=[rows(dx), rows(dx), rows(_H), rows(_H),
                  whole(W0x), whole(W1x), whole(bx),
                  whole(W0h), whole(W1h), whole(bh),
                  whole(W0hh), whole(bhh)],
        out_specs=[rows(_H), rows(_H), rows(_H)],
        out_shape=[o64, o64, o64],
    )(X, Px, H, Ph, W0x, W1x, bx, W0h, W1h, bh, W0hh, bhh)


def _update_tc(Z, H, S, Phr, W1hh, LW, Lb):
    def body(z_ref, h_ref, s_ref, phr_ref, w1hh_ref, lw_ref, lb_ref,
             hn_ref, o_ref):
        Ht = jnp.tanh(s_ref[...] + phr_ref[...] @ w1hh_ref[...])
        Hn = z_ref[...] * h_ref[...] + (1.0 - z_ref[...]) * Ht
        hn_ref[...] = Hn
        o_ref[...] = jnp.maximum(Hn, 0.0) @ lw_ref[...] + lb_ref[...]

    dL = LW.shape[1]
    grid = (_N // _BLK,)
    rows = lambda d: pl.BlockSpec((_BLK, d), lambda i: (i, 0))
    whole = lambda a: pl.BlockSpec(a.shape, lambda i: (0,) * a.ndim)
    return pl.pallas_call(
        body,
        grid=grid,
        in_specs=[rows(_H), rows(_H), rows(_H), rows(_H),
                  whole(W1hh), whole(LW), whole(Lb)],
        out_specs=[rows(_H), rows(dL)],
        out_shape=[jax.ShapeDtypeStruct((_N, _H), jnp.float32),
                   jax.ShapeDtypeStruct((_N, dL), jnp.float32)],
    )(Z, H, S, Phr, W1hh, LW, Lb)


def _cat(p, pre, names, key, axis=1):
    return jnp.concatenate([p[pre + nm + key] for nm in names], axis=axis)


def kernel(x_seq, edge_index, edge_weight, h_enc, h_dec, params):
    row, col = edge_index[0], edge_index[1]

    perm = jnp.argsort(col).astype(jnp.int32)
    row_s = row[perm]
    col_s = col[perm]
    eo = jnp.searchsorted(
        col_s, jnp.arange(_NPART + 1, dtype=jnp.int32) * _PR).astype(jnp.int32)
    eo_pad = jnp.concatenate([eo, jnp.zeros((80 - _NPART - 1,), jnp.int32)])
    crel = (col_s % _PR).astype(jnp.int32)
    zpad = jnp.zeros((_EPAD - _E,), jnp.int32)
    row_sp = jnp.concatenate([row_s, zpad])
    crel_p = jnp.concatenate([crel, zpad])

    deg2 = _deg_kernel(row, edge_weight)
    dis = _dis_tc(deg2)
    lhat_p = _lhat_kernel(perm, row_s, col_s, edge_weight, dis)

    def prop(table):
        D = table.shape[1]
        out = _make_prop(D)(table, row_sp, crel_p, lhat_p, eo_pad)
        return out[:_N]

    p = params
    xp_seq = jnp.zeros((_T, _N, 16), jnp.float32).at[:, :, :4].set(x_seq)
    zrh = ['xz', 'xr', 'xh']
    hzr = ['hz', 'hr']
    wx_e0 = jnp.zeros((16, 192), jnp.float32).at[:4].set(_cat(p, 'enc_', zrh, '_W0'))
    wx_e1 = jnp.zeros((16, 192), jnp.float32).at[:4].set(_cat(p, 'enc_', zrh, '_W1'))
    bx_e = _cat(p, 'enc_', zrh, '_b', axis=0).reshape(1, 192)
    wh_e0 = _cat(p, 'enc_', hzr, '_W0')
    wh_e1 = _cat(p, 'enc_', hzr, '_W1')
    bh_e = _cat(p, 'enc_', hzr, '_b', axis=0).reshape(1, 128)
    wx_d0 = _cat(p, 'dec_', zrh, '_W0')
    wx_d1 = _cat(p, 'dec_', zrh, '_W1')
    bx_d = _cat(p, 'dec_', zrh, '_b', axis=0).reshape(1, 192)
    wh_d0 = _cat(p, 'dec_', hzr, '_W0')
    wh_d1 = _cat(p, 'dec_', hzr, '_W1')
    bh_d = _cat(p, 'dec_', hzr, '_b', axis=0).reshape(1, 128)
    bhh_e = p['enc_hh_b'].reshape(1, _H)
    bhh_d = p['dec_hh_b'].reshape(1, _H)
    elb = p['enc_lin_b'].reshape(1, -1)
    dlb = p['dec_lin_b'].reshape(1, -1)

    y = None
    for t in range(_T):
        xp = xp_seq[t]
        Px = prop(xp)
        Ph = prop(h_enc)
        Z, HR, S = _gates_tc(xp, Px, h_enc, Ph, wx_e0, wx_e1, bx_e,
                             wh_e0, wh_e1, bh_e, p['enc_hh_W0'], bhh_e)
        Phr = prop(HR)
        h_enc, z = _update_tc(Z, h_enc, S, Phr, p['enc_hh_W1'],
                              p['enc_lin_W'], elb)
        Pz = prop(z)
        Phd = prop(h_dec)
        Z2, HR2, S2 = _gates_tc(z, Pz, h_dec, Phd, wx_d0, wx_d1, bx_d,
                                wh_d0, wh_d1, bh_d, p['dec_hh_W0'], bhh_d)
        Phdr = prop(HR2)
        h_dec, y = _update_tc(Z2, h_dec, S2, Phdr, p['dec_hh_W1'],
                              p['dec_lin_W'], dlb)
    return y


# speedup vs baseline: 1.5776x; 1.0283x over previous
import functools

import jax
import jax.numpy as jnp
from jax import lax
from jax.experimental import pallas as pl
from jax.experimental.pallas import tpu as pltpu
from jax.experimental.pallas import tpu_sc as plsc

_N = 50000
_E = 1600000
_T = 4
_H = 64

_NC = 2
_NS = 16
_NW = _NC * _NS
_EPW = _E // _NW

_NPART = 64
_PR = 784
_NPAD = _NPART * _PR

_CH = 512
_CHSH = 9
_EPAD = _E + 2 * _CH
_CH2 = 2000

_BLK = 2000

_mesh = plsc.VectorSubcoreMesh(core_axis_name="c", subcore_axis_name="s")
_SC_PARAMS = pltpu.CompilerParams(use_tc_tiling_on_sc=False,
                                  needs_layout_passes=False)


@functools.partial(
    pl.kernel, mesh=_mesh, compiler_params=_SC_PARAMS,
    out_type=jax.ShapeDtypeStruct((_NC, _NPAD), jnp.float32),
    scratch_types=[
        pltpu.VMEM((_CH2,), jnp.int32),
        pltpu.VMEM((_CH2,), jnp.float32),
        pltpu.VMEM((_NPAD // _NS,), jnp.float32),
        pltpu.VMEM_SHARED((_NPAD,), jnp.float32),
    ],
)
def _deg_kernel(rows_hbm, w_hbm, out, ibuf, wbuf, zbuf, shacc):
    cid = lax.axis_index("c")
    sid = lax.axis_index("s")
    zero = jnp.zeros((16,), jnp.float32)
    zn = _NPAD // _NS

    def zb(j, c):
        zbuf[pl.ds(j * 16, 16)] = zero
        return c
    lax.fori_loop(0, zn // 16, zb, 0)
    pltpu.sync_copy(zbuf, shacc.at[pl.ds(pl.multiple_of(sid * zn, 8), zn)])
    plsc.subcore_barrier()

    wid = sid * _NC + cid
    base = wid * _EPW
    for k in range(_EPW // _CH2):
        s = base + k * _CH2
        pltpu.sync_copy(rows_hbm.at[pl.ds(s, _CH2)], ibuf)
        pltpu.sync_copy(w_hbm.at[pl.ds(s, _CH2)], wbuf)
        pltpu.sync_copy(wbuf, shacc.at[ibuf], add=True)
    plsc.subcore_barrier()

    @pl.when(sid == 0)
    def _():
        pltpu.sync_copy(shacc, out.at[cid])


@functools.partial(
    pl.kernel, mesh=_mesh, compiler_params=_SC_PARAMS,
    out_type=jax.ShapeDtypeStruct((_EPAD,), jnp.float32),
    scratch_types=[
        pltpu.VMEM((_CH2,), jnp.int32),
        pltpu.VMEM((_CH2,), jnp.int32),
        pltpu.VMEM((_CH2,), jnp.int32),
        pltpu.VMEM((_CH2,), jnp.float32),
        pltpu.VMEM((_CH2,), jnp.float32),
        pltpu.VMEM((_CH2,), jnp.float32),
        pltpu.VMEM((_CH2,), jnp.float32),
        pltpu.SemaphoreType.DMA,
    ],
)
def _lhat_kernel(perm_hbm, rows_hbm, cols_hbm, ew_hbm, dis_hbm, out,
                 pbuf, rbuf, cbuf, wbuf, drbuf, dcbuf, obuf, sem):
    cid = lax.axis_index("c")
    sid = lax.axis_index("s")
    wid = sid * _NC + cid
    base = wid * _EPW
    for k in range(_EPW // _CH2):
        s = base + k * _CH2
        pltpu.sync_copy(perm_hbm.at[pl.ds(s, _CH2)], pbuf)
        pltpu.sync_copy(rows_hbm.at[pl.ds(s, _CH2)], rbuf)
        pltpu.sync_copy(cols_hbm.at[pl.ds(s, _CH2)], cbuf)
        pltpu.async_copy(ew_hbm.at[pbuf], wbuf, sem).wait()
        pltpu.async_copy(dis_hbm.at[rbuf], drbuf, sem).wait()
        pltpu.async_copy(dis_hbm.at[cbuf], dcbuf, sem).wait()

        def vb(j, c):
            sl = pl.ds(j * 16, 16)
            obuf[sl] = -(drbuf[sl] * wbuf[sl] * dcbuf[sl])
            return c
        lax.fori_loop(0, _CH2 // 16, vb, 0)
        pltpu.sync_copy(obuf, out.at[pl.ds(s, _CH2)])


def _vextract(vec_ref, p):
    base = pl.multiple_of((p >> 4) << 4, 16)
    v = vec_ref[pl.ds(base, 16)]
    iota = lax.broadcasted_iota(jnp.int32, (16,), 0)
    return jnp.sum(jnp.where(iota == (p & 15), v, 0))


@functools.cache
def _make_prop(D):
    @functools.partial(
        pl.kernel, mesh=_mesh, compiler_params=_SC_PARAMS,
        out_type=jax.ShapeDtypeStruct((_NPAD, D), jnp.float32),
        scratch_types=(
            [pltpu.VMEM((_PR, 16), jnp.float32)] * (D // 16) + [
                pltpu.VMEM((_CH,), jnp.int32),
                pltpu.VMEM((_CH,), jnp.int32),
                pltpu.VMEM((_CH,), jnp.float32),
                pltpu.VMEM((80,), jnp.int32),
                pltpu.VMEM((_CH, D), jnp.float32),
                pltpu.SemaphoreType.DMA,
            ]
        ),
    )
    def _prop_kernel(table, rows_hbm, crel_hbm, lhat_hbm, eo_hbm, out, *scr):
        nq = D // 16
        accs = scr[:nq]
        ibuf, cbuf, lbuf, eov, gbuf, sem = scr[nq:]
        cid = lax.axis_index("c")
        sid = lax.axis_index("s")
        wid = sid * _NC + cid
        pltpu.sync_copy(eo_hbm, eov)
        zero = jnp.zeros((16,), jnp.float32)
        iota = lax.broadcasted_iota(jnp.int32, (16,), 0)
        for sub in range(2):
            p = wid * 2 + sub

            def zb(j, c):
                for q in range(nq):
                    accs[q][j, pl.ds(0, 16)] = zero
                return c
            lax.fori_loop(0, _PR, zb, 0)

            e0 = _vextract(eov, p)
            e1 = _vextract(eov, p + 1)
            ab = (e0 >> 3) << 3
            nch = (e1 - ab + _CH - 1) >> _CHSH

            def chunk(k, c):
                ck = pl.multiple_of(ab + k * _CH, 8)
                pltpu.sync_copy(rows_hbm.at[pl.ds(ck, _CH)], ibuf)
                pltpu.sync_copy(crel_hbm.at[pl.ds(ck, _CH)], cbuf)
                pltpu.sync_copy(lhat_hbm.at[pl.ds(ck, _CH)], lbuf)
                pltpu.async_copy(table.at[ibuf], gbuf, sem).wait()
                lo = e0 - ck
                hi = e1 - ck

                def eb(g, cc):
                    b = g * 16
                    lane = b + iota
                    m = (lane >= lo) & (lane < hi)
                    crv = cbuf[pl.ds(b, 16)]
                    wv = lbuf[pl.ds(b, 16)]
                    rowi = lane
                    for l in range(16):
                        li = jnp.full((16,), l, jnp.int32)
                        for q in range(nq):
                            gl = jnp.full((16,), q * 16 + l, jnp.int32)
                            colv = plsc.load_gather(gbuf, [rowi, gl])
                            plsc.addupdate_scatter(accs[q], [crv, li],
                                                   colv * wv, mask=m)
                    return cc
                lax.fori_loop(0, _CH // 16, eb, 0)
                return c
            lax.fori_loop(0, nch, chunk, 0)
            ro = pl.ds(pl.multiple_of(p * _PR, 8), _PR)
            for q in range(nq):
                pltpu.sync_copy(accs[q], out.at[ro, pl.ds(q * 16, 16)])

    return _prop_kernel


def _dis_tc(deg2):
    def body(a_ref, o_ref):
        d = a_ref[0] + a_ref[1]
        o_ref[...] = jnp.where(d > 0, lax.rsqrt(jnp.where(d > 0, d, 1.0)), 0.0)

    out = pl.pallas_call(
        body,
        out_shape=jax.ShapeDtypeStruct((_NPAD // 128, 128), jnp.float32),
    )(deg2.reshape(2, _NPAD // 128, 128))
    return out.reshape(_NPAD)


def _gates_tc(X, Px, H, Ph, W0x, W1x, bx, W0h, W1h, bh, W0hh, bhh):
    def body(x_ref, px_ref, h_ref, ph_ref, w0x_ref, w1x_ref, bx_ref,
             w0h_ref, w1h_ref, bh_ref, w0hh_ref, bhh_ref,
             z_ref, hr_ref, s_ref):
        gx = (x_ref[...] @ w0x_ref[...] + px_ref[...] @ w1x_ref[...]
              + bx_ref[...])
        gh = (h_ref[...] @ w0h_ref[...] + ph_ref[...] @ w1h_ref[...]
              + bh_ref[...])
        Z = jax.nn.sigmoid(gx[:, :_H] + gh[:, :_H])
        R = jax.nn.sigmoid(gx[:, _H:2 * _H] + gh[:, _H:])
        HR = h_ref[...] * R
        z_ref[...] = Z
        hr_ref[...] = HR
        s_ref[...] = gx[:, 2 * _H:] + HR @ w0hh_ref[...] + bhh_ref[...]

    dx = X.shape[1]
    grid = (_N // _BLK,)
    rows = lambda d: pl.BlockSpec((_BLK, d), lambda i: (i, 0))
    whole = lambda a: pl.BlockSpec(a.shape, lambda i: (0,) * a.ndim)
    o64 = jax.ShapeDtypeStruct((_N, _H), jnp.float32)
    return pl.pallas_call(
        body,
        grid=grid,
        in_specs=[rows(dx), rows(dx), rows(_H), rows(_H),
                  whole(W0x), whole(W1x), whole(bx),
                  whole(W0h), whole(W1h), whole(bh),
                  whole(W0hh), whole(bhh)],
        out_specs=[rows(_H), rows(_H), rows(_H)],
        out_shape=[o64, o64, o64],
    )(X, Px, H, Ph, W0x, W1x, bx, W0h, W1h, bh, W0hh, bhh)


def _update_tc(Z, H, S, Phr, W1hh, LW, Lb):
    def body(z_ref, h_ref, s_ref, phr_ref, w1hh_ref, lw_ref, lb_ref,
             hn_ref, o_ref):
        Ht = jnp.tanh(s_ref[...] + phr_ref[...] @ w1hh_ref[...])
        Hn = z_ref[...] * h_ref[...] + (1.0 - z_ref[...]) * Ht
        hn_ref[...] = Hn
        o_ref[...] = jnp.maximum(Hn, 0.0) @ lw_ref[...] + lb_ref[...]

    dL = LW.shape[1]
    grid = (_N // _BLK,)
    rows = lambda d: pl.BlockSpec((_BLK, d), lambda i: (i, 0))
    whole = lambda a: pl.BlockSpec(a.shape, lambda i: (0,) * a.ndim)
    return pl.pallas_call(
        body,
        grid=grid,
        in_specs=[rows(_H), rows(_H), rows(_H), rows(_H),
                  whole(W1hh), whole(LW), whole(Lb)],
        out_specs=[rows(_H), rows(dL)],
        out_shape=[jax.ShapeDtypeStruct((_N, _H), jnp.float32),
                   jax.ShapeDtypeStruct((_N, dL), jnp.float32)],
    )(Z, H, S, Phr, W1hh, LW, Lb)


def _cat(p, pre, names, key, axis=1):
    return jnp.concatenate([p[pre + nm + key] for nm in names], axis=axis)


def kernel(x_seq, edge_index, edge_weight, h_enc, h_dec, params):
    row, col = edge_index[0], edge_index[1]

    perm = jnp.argsort(col).astype(jnp.int32)
    row_s = row[perm]
    col_s = col[perm]
    eo = jnp.searchsorted(
        col_s, jnp.arange(_NPART + 1, dtype=jnp.int32) * _PR).astype(jnp.int32)
    eo_pad = jnp.concatenate([eo, jnp.zeros((80 - _NPART - 1,), jnp.int32)])
    crel = (col_s % _PR).astype(jnp.int32)
    zpad = jnp.zeros((_EPAD - _E,), jnp.int32)
    row_sp = jnp.concatenate([row_s, zpad])
    crel_p = jnp.concatenate([crel, zpad])

    deg2 = _deg_kernel(row, edge_weight)
    dis = _dis_tc(deg2)
    lhat_p = _lhat_kernel(perm, row_s, col_s, edge_weight, dis)

    def prop(table):
        D = table.shape[1]
        out = _make_prop(D)(table, row_sp, crel_p, lhat_p, eo_pad)
        return out[:_N]

    p = params
    xp_seq = jnp.zeros((_T, _N, 16), jnp.float32).at[:, :, :4].set(x_seq)
    zrh = ['xz', 'xr', 'xh']
    hzr = ['hz', 'hr']
    wx_e0 = jnp.zeros((16, 192), jnp.float32).at[:4].set(_cat(p, 'enc_', zrh, '_W0'))
    wx_e1 = jnp.zeros((16, 192), jnp.float32).at[:4].set(_cat(p, 'enc_', zrh, '_W1'))
    bx_e = _cat(p, 'enc_', zrh, '_b', axis=0).reshape(1, 192)
    wh_e0 = _cat(p, 'enc_', hzr, '_W0')
    wh_e1 = _cat(p, 'enc_', hzr, '_W1')
    bh_e = _cat(p, 'enc_', hzr, '_b', axis=0).reshape(1, 128)
    wx_d0 = _cat(p, 'dec_', zrh, '_W0')
    wx_d1 = _cat(p, 'dec_', zrh, '_W1')
    bx_d = _cat(p, 'dec_', zrh, '_b', axis=0).reshape(1, 192)
    wh_d0 = _cat(p, 'dec_', hzr, '_W0')
    wh_d1 = _cat(p, 'dec_', hzr, '_W1')
    bh_d = _cat(p, 'dec_', hzr, '_b', axis=0).reshape(1, 128)
    bhh_e = p['enc_hh_b'].reshape(1, _H)
    bhh_d = p['dec_hh_b'].reshape(1, _H)
    elb = p['enc_lin_b'].reshape(1, -1)
    dlb = p['dec_lin_b'].reshape(1, -1)

    y = None
    for t in range(_T):
        xp = xp_seq[t]
        Px = prop(xp)
        Ph = prop(h_enc)
        Z, HR, S = _gates_tc(xp, Px, h_enc, Ph, wx_e0, wx_e1, bx_e,
                             wh_e0, wh_e1, bh_e, p['enc_hh_W0'], bhh_e)
        Phr = prop(HR)
        h_enc, z = _update_tc(Z, h_enc, S, Phr, p['enc_hh_W1'],
                              p['enc_lin_W'], elb)
        Pz = prop(z)
        Phd = prop(h_dec)
        Z2, HR2, S2 = _gates_tc(z, Pz, h_dec, Phd, wx_d0, wx_d1, bx_d,
                                wh_d0, wh_d1, bh_d, p['dec_hh_W0'], bhh_d)
        Phdr = prop(HR2)
        h_dec, y = _update_tc(Z2, h_dec, S2, Phdr, p['dec_hh_W1'],
                              p['dec_lin_W'], dlb)
    return y


# speedup vs baseline: 2.0228x; 1.2822x over previous
import functools

import jax
import jax.numpy as jnp
from jax import lax
from jax.experimental import pallas as pl
from jax.experimental.pallas import tpu as pltpu
from jax.experimental.pallas import tpu_sc as plsc

_N = 50000
_E = 1600000
_T = 4
_H = 64

_NC = 2
_NS = 16
_NW = _NC * _NS
_EPW = _E // _NW

_NPART = 64
_PR = 784
_NPAD = _NPART * _PR

_CH = 512
_CHSH = 9
_EPAD = _E + 2 * _CH
_CH2 = 2000

_BLK = 2000

_mesh = plsc.VectorSubcoreMesh(core_axis_name="c", subcore_axis_name="s")
_SC_PARAMS = pltpu.CompilerParams(use_tc_tiling_on_sc=False,
                                  needs_layout_passes=False)


@functools.partial(
    pl.kernel, mesh=_mesh, compiler_params=_SC_PARAMS,
    out_type=jax.ShapeDtypeStruct((_NC, _NPAD), jnp.float32),
    scratch_types=[
        pltpu.VMEM((_CH2,), jnp.int32),
        pltpu.VMEM((_CH2,), jnp.float32),
        pltpu.VMEM((_NPAD // _NS,), jnp.float32),
        pltpu.VMEM_SHARED((_NPAD,), jnp.float32),
    ],
)
def _deg_kernel(rows_hbm, w_hbm, out, ibuf, wbuf, zbuf, shacc):
    cid = lax.axis_index("c")
    sid = lax.axis_index("s")
    zero = jnp.zeros((16,), jnp.float32)
    zn = _NPAD // _NS

    def zb(j, c):
        zbuf[pl.ds(j * 16, 16)] = zero
        return c
    lax.fori_loop(0, zn // 16, zb, 0)
    pltpu.sync_copy(zbuf, shacc.at[pl.ds(pl.multiple_of(sid * zn, 8), zn)])
    plsc.subcore_barrier()

    wid = sid * _NC + cid
    base = wid * _EPW
    for k in range(_EPW // _CH2):
        s = base + k * _CH2
        pltpu.sync_copy(rows_hbm.at[pl.ds(s, _CH2)], ibuf)
        pltpu.sync_copy(w_hbm.at[pl.ds(s, _CH2)], wbuf)
        pltpu.sync_copy(wbuf, shacc.at[ibuf], add=True)
    plsc.subcore_barrier()

    @pl.when(sid == 0)
    def _():
        pltpu.sync_copy(shacc, out.at[cid])


@functools.partial(
    pl.kernel, mesh=_mesh, compiler_params=_SC_PARAMS,
    out_type=jax.ShapeDtypeStruct((_EPAD,), jnp.float32),
    scratch_types=[
        pltpu.VMEM((_CH2,), jnp.int32),
        pltpu.VMEM((_CH2,), jnp.int32),
        pltpu.VMEM((_CH2,), jnp.int32),
        pltpu.VMEM((_CH2,), jnp.float32),
        pltpu.VMEM((_CH2,), jnp.float32),
        pltpu.VMEM((_CH2,), jnp.float32),
        pltpu.VMEM((_CH2,), jnp.float32),
        pltpu.SemaphoreType.DMA,
    ],
)
def _lhat_kernel(perm_hbm, rows_hbm, cols_hbm, ew_hbm, dis_hbm, out,
                 pbuf, rbuf, cbuf, wbuf, drbuf, dcbuf, obuf, sem):
    cid = lax.axis_index("c")
    sid = lax.axis_index("s")
    wid = sid * _NC + cid
    base = wid * _EPW
    for k in range(_EPW // _CH2):
        s = base + k * _CH2
        pltpu.sync_copy(perm_hbm.at[pl.ds(s, _CH2)], pbuf)
        pltpu.sync_copy(rows_hbm.at[pl.ds(s, _CH2)], rbuf)
        pltpu.sync_copy(cols_hbm.at[pl.ds(s, _CH2)], cbuf)
        pltpu.async_copy(ew_hbm.at[pbuf], wbuf, sem).wait()
        pltpu.async_copy(dis_hbm.at[rbuf], drbuf, sem).wait()
        pltpu.async_copy(dis_hbm.at[cbuf], dcbuf, sem).wait()

        def vb(j, c):
            sl = pl.ds(j * 16, 16)
            obuf[sl] = -(drbuf[sl] * wbuf[sl] * dcbuf[sl])
            return c
        lax.fori_loop(0, _CH2 // 16, vb, 0)
        pltpu.sync_copy(obuf, out.at[pl.ds(s, _CH2)])


def _vextract(vec_ref, p):
    base = pl.multiple_of((p >> 4) << 4, 16)
    v = vec_ref[pl.ds(base, 16)]
    iota = lax.broadcasted_iota(jnp.int32, (16,), 0)
    return jnp.sum(jnp.where(iota == (p & 15), v, 0))


@functools.cache
def _make_prop(D):
    @functools.partial(
        pl.kernel, mesh=_mesh, compiler_params=_SC_PARAMS,
        out_type=jax.ShapeDtypeStruct((_NPAD, D), jnp.float32),
        scratch_types=(
            [pltpu.VMEM((_PR, 16), jnp.float32)] * (D // 16) + [
                pltpu.VMEM((_CH,), jnp.int32),
                pltpu.VMEM((_CH,), jnp.int32),
                pltpu.VMEM((_CH,), jnp.float32),
                pltpu.VMEM((80,), jnp.int32),
                pltpu.VMEM((_CH, D), jnp.float32),
                pltpu.SemaphoreType.DMA,
            ]
        ),
    )
    def _prop_kernel(table, rows_hbm, crel_hbm, lhat_hbm, eo_hbm, out, *scr):
        nq = D // 16
        accs = scr[:nq]
        ibuf, cbuf, lbuf, eov, gbuf, sem = scr[nq:]
        cid = lax.axis_index("c")
        sid = lax.axis_index("s")
        wid = sid * _NC + cid
        pltpu.sync_copy(eo_hbm, eov)
        zero = jnp.zeros((16,), jnp.float32)
        iota = lax.broadcasted_iota(jnp.int32, (16,), 0)
        for sub in range(2):
            p = wid * 2 + sub

            def zb(j, c):
                for q in range(nq):
                    accs[q][j, pl.ds(0, 16)] = zero
                return c
            lax.fori_loop(0, _PR, zb, 0)

            e0 = _vextract(eov, p)
            e1 = _vextract(eov, p + 1)
            ab = (e0 >> 3) << 3
            nch = (e1 - ab + _CH - 1) >> _CHSH

            def chunk(k, c):
                ck = pl.multiple_of(ab + k * _CH, 8)
                pltpu.sync_copy(rows_hbm.at[pl.ds(ck, _CH)], ibuf)
                pltpu.sync_copy(crel_hbm.at[pl.ds(ck, _CH)], cbuf)
                pltpu.sync_copy(lhat_hbm.at[pl.ds(ck, _CH)], lbuf)
                pltpu.async_copy(table.at[ibuf], gbuf, sem).wait()
                lo = e0 - ck
                hi = e1 - ck

                def eb(g, cc):
                    b = g * 16
                    lane = b + iota
                    m = (lane >= lo) & (lane < hi)
                    crv = cbuf[pl.ds(b, 16)]
                    wv = lbuf[pl.ds(b, 16)]
                    rowi = lane
                    for l in range(16):
                        li = jnp.full((16,), l, jnp.int32)
                        for q in range(nq):
                            gl = jnp.full((16,), q * 16 + l, jnp.int32)
                            colv = plsc.load_gather(gbuf, [rowi, gl])
                            plsc.addupdate_scatter(accs[q], [crv, li],
                                                   colv * wv, mask=m)
                    return cc
                lax.fori_loop(0, _CH // 16, eb, 0)
                return c
            lax.fori_loop(0, nch, chunk, 0)
            ro = pl.ds(pl.multiple_of(p * _PR, 8), _PR)
            for q in range(nq):
                pltpu.sync_copy(accs[q], out.at[ro, pl.ds(q * 16, 16)])

    return _prop_kernel


def _dis_tc(deg2):
    def body(a_ref, o_ref):
        d = a_ref[0] + a_ref[1]
        o_ref[...] = jnp.where(d > 0, lax.rsqrt(jnp.where(d > 0, d, 1.0)), 0.0)

    out = pl.pallas_call(
        body,
        out_shape=jax.ShapeDtypeStruct((_NPAD // 128, 128), jnp.float32),
    )(deg2.reshape(2, _NPAD // 128, 128))
    return out.reshape(_NPAD)


def _gates_tc(X, Px, H, Ph, W0x, W1x, bx, W0h, W1h, bh, W0hh, bhh):
    def body(x_ref, px_ref, h_ref, ph_ref, w0x_ref, w1x_ref, bx_ref,
             w0h_ref, w1h_ref, bh_ref, w0hh_ref, bhh_ref,
             z_ref, hr_ref, s_ref):
        gx = (x_ref[...] @ w0x_ref[...] + px_ref[...] @ w1x_ref[...]
              + bx_ref[...])
        gh = (h_ref[...] @ w0h_ref[...] + ph_ref[...] @ w1h_ref[...]
              + bh_ref[...])
        Z = jax.nn.sigmoid(gx[:, :_H] + gh[:, :_H])
        R = jax.nn.sigmoid(gx[:, _H:2 * _H] + gh[:, _H:])
        HR = h_ref[...] * R
        z_ref[...] = Z
        hr_ref[...] = HR
        s_ref[...] = gx[:, 2 * _H:] + HR @ w0hh_ref[...] + bhh_ref[...]

    dx = X.shape[1]
    grid = (_N // _BLK,)
    rows = lambda d: pl.BlockSpec((_BLK, d), lambda i: (i, 0))
    whole = lambda a: pl.BlockSpec(a.shape, lambda i: (0,) * a.ndim)
    o64 = jax.ShapeDtypeStruct((_N, _H), jnp.float32)
    return pl.pallas_call(
        body,
        grid=grid,
        in_specs=[rows(dx), rows(dx), rows(_H), rows(_H),
                  whole(W0x), whole(W1x), whole(bx),
                  whole(W0h), whole(W1h), whole(bh),
                  whole(W0hh), whole(bhh)],
        out_specs=[rows(_H), rows(_H), rows(_H)],
        out_shape=[o64, o64, o64],
    )(X, Px, H, Ph, W0x, W1x, bx, W0h, W1h, bh, W0hh, bhh)


def _update_tc(Z, H, S, Phr, W1hh, LW, Lb):
    def body(z_ref, h_ref, s_ref, phr_ref, w1hh_ref, lw_ref, lb_ref,
             hn_ref, o_ref):
        Ht = jnp.tanh(s_ref[...] + phr_ref[...] @ w1hh_ref[...])
        Hn = z_ref[...] * h_ref[...] + (1.0 - z_ref[...]) * Ht
        hn_ref[...] = Hn
        o_ref[...] = jnp.maximum(Hn, 0.0) @ lw_ref[...] + lb_ref[...]

    dL = LW.shape[1]
    grid = (_N // _BLK,)
    rows = lambda d: pl.BlockSpec((_BLK, d), lambda i: (i, 0))
    whole = lambda a: pl.BlockSpec(a.shape, lambda i: (0,) * a.ndim)
    return pl.pallas_call(
        body,
        grid=grid,
        in_specs=[rows(_H), rows(_H), rows(_H), rows(_H),
                  whole(W1hh), whole(LW), whole(Lb)],
        out_specs=[rows(_H), rows(dL)],
        out_shape=[jax.ShapeDtypeStruct((_N, _H), jnp.float32),
                   jax.ShapeDtypeStruct((_N, dL), jnp.float32)],
    )(Z, H, S, Phr, W1hh, LW, Lb)


def _cat(p, pre, names, key, axis=1):
    return jnp.concatenate([p[pre + nm + key] for nm in names], axis=axis)


def kernel(x_seq, edge_index, edge_weight, h_enc, h_dec, params):
    row, col = edge_index[0], edge_index[1]

    perm = jnp.argsort(col).astype(jnp.int32)
    part = col[perm] // _PR
    key = part * 64 + (jnp.arange(_E, dtype=jnp.int32) % 64)
    perm2 = jnp.argsort(key, stable=True).astype(jnp.int32)
    perm = perm[perm2]
    part_s = part[perm2]
    row_s = row[perm]
    col_s = col[perm]
    eo = jnp.searchsorted(
        part_s, jnp.arange(_NPART + 1, dtype=jnp.int32)).astype(jnp.int32)
    eo_pad = jnp.concatenate([eo, jnp.zeros((80 - _NPART - 1,), jnp.int32)])
    crel = (col_s % _PR).astype(jnp.int32)
    zpad = jnp.zeros((_EPAD - _E,), jnp.int32)
    row_sp = jnp.concatenate([row_s, zpad])
    crel_p = jnp.concatenate([crel, zpad])

    deg2 = _deg_kernel(row, edge_weight)
    dis = _dis_tc(deg2)
    lhat_p = _lhat_kernel(perm, row_s, col_s, edge_weight, dis)

    def prop(table):
        D = table.shape[1]
        out = _make_prop(D)(table, row_sp, crel_p, lhat_p, eo_pad)
        return out[:_N]

    p = params
    xp_seq = jnp.zeros((_T, _N, 16), jnp.float32).at[:, :, :4].set(x_seq)
    zrh = ['xz', 'xr', 'xh']
    hzr = ['hz', 'hr']
    wx_e0 = jnp.zeros((16, 192), jnp.float32).at[:4].set(_cat(p, 'enc_', zrh, '_W0'))
    wx_e1 = jnp.zeros((16, 192), jnp.float32).at[:4].set(_cat(p, 'enc_', zrh, '_W1'))
    bx_e = _cat(p, 'enc_', zrh, '_b', axis=0).reshape(1, 192)
    wh_e0 = _cat(p, 'enc_', hzr, '_W0')
    wh_e1 = _cat(p, 'enc_', hzr, '_W1')
    bh_e = _cat(p, 'enc_', hzr, '_b', axis=0).reshape(1, 128)
    wx_d0 = _cat(p, 'dec_', zrh, '_W0')
    wx_d1 = _cat(p, 'dec_', zrh, '_W1')
    bx_d = _cat(p, 'dec_', zrh, '_b', axis=0).reshape(1, 192)
    wh_d0 = _cat(p, 'dec_', hzr, '_W0')
    wh_d1 = _cat(p, 'dec_', hzr, '_W1')
    bh_d = _cat(p, 'dec_', hzr, '_b', axis=0).reshape(1, 128)
    bhh_e = p['enc_hh_b'].reshape(1, _H)
    bhh_d = p['dec_hh_b'].reshape(1, _H)
    elb = p['enc_lin_b'].reshape(1, -1)
    dlb = p['dec_lin_b'].reshape(1, -1)

    y = None
    for t in range(_T):
        xp = xp_seq[t]
        Px = prop(xp)
        Ph = prop(h_enc)
        Z, HR, S = _gates_tc(xp, Px, h_enc, Ph, wx_e0, wx_e1, bx_e,
                             wh_e0, wh_e1, bh_e, p['enc_hh_W0'], bhh_e)
        Phr = prop(HR)
        h_enc, z = _update_tc(Z, h_enc, S, Phr, p['enc_hh_W1'],
                              p['enc_lin_W'], elb)
        Pz = prop(z)
        Phd = prop(h_dec)
        Z2, HR2, S2 = _gates_tc(z, Pz, h_dec, Phd, wx_d0, wx_d1, bx_d,
                                wh_d0, wh_d1, bh_d, p['dec_hh_W0'], bhh_d)
        Phdr = prop(HR2)
        h_dec, y = _update_tc(Z2, h_dec, S2, Phdr, p['dec_hh_W1'],
                              p['dec_lin_W'], dlb)
    return y


# speedup vs baseline: 2.0688x; 1.0227x over previous
import functools

import jax
import jax.numpy as jnp
from jax import lax
from jax.experimental import pallas as pl
from jax.experimental.pallas import tpu as pltpu
from jax.experimental.pallas import tpu_sc as plsc

_N = 50000
_E = 1600000
_T = 4
_H = 64

_NC = 2
_NS = 16
_NW = _NC * _NS
_EPW = _E // _NW

_NPART = 64
_PR = 784
_NPAD = _NPART * _PR

_CH = 1024
_CHSH = 10
_EPAD = _E + 2 * _CH
_CH2 = 2000

_BLK = 2000

_mesh = plsc.VectorSubcoreMesh(core_axis_name="c", subcore_axis_name="s")
_SC_PARAMS = pltpu.CompilerParams(use_tc_tiling_on_sc=False,
                                  needs_layout_passes=False)


@functools.partial(
    pl.kernel, mesh=_mesh, compiler_params=_SC_PARAMS,
    out_type=jax.ShapeDtypeStruct((_NC, _NPAD), jnp.float32),
    scratch_types=[
        pltpu.VMEM((_CH2,), jnp.int32),
        pltpu.VMEM((_CH2,), jnp.float32),
        pltpu.VMEM((_NPAD // _NS,), jnp.float32),
        pltpu.VMEM_SHARED((_NPAD,), jnp.float32),
    ],
)
def _deg_kernel(rows_hbm, w_hbm, out, ibuf, wbuf, zbuf, shacc):
    cid = lax.axis_index("c")
    sid = lax.axis_index("s")
    zero = jnp.zeros((16,), jnp.float32)
    zn = _NPAD // _NS

    def zb(j, c):
        zbuf[pl.ds(j * 16, 16)] = zero
        return c
    lax.fori_loop(0, zn // 16, zb, 0)
    pltpu.sync_copy(zbuf, shacc.at[pl.ds(pl.multiple_of(sid * zn, 8), zn)])
    plsc.subcore_barrier()

    wid = sid * _NC + cid
    base = wid * _EPW
    for k in range(_EPW // _CH2):
        s = base + k * _CH2
        pltpu.sync_copy(rows_hbm.at[pl.ds(s, _CH2)], ibuf)
        pltpu.sync_copy(w_hbm.at[pl.ds(s, _CH2)], wbuf)
        pltpu.sync_copy(wbuf, shacc.at[ibuf], add=True)
    plsc.subcore_barrier()

    @pl.when(sid == 0)
    def _():
        pltpu.sync_copy(shacc, out.at[cid])


@functools.partial(
    pl.kernel, mesh=_mesh, compiler_params=_SC_PARAMS,
    out_type=jax.ShapeDtypeStruct((_EPAD,), jnp.float32),
    scratch_types=[
        pltpu.VMEM((_CH2,), jnp.int32),
        pltpu.VMEM((_CH2,), jnp.int32),
        pltpu.VMEM((_CH2,), jnp.int32),
        pltpu.VMEM((_CH2,), jnp.float32),
        pltpu.VMEM((_CH2,), jnp.float32),
        pltpu.VMEM((_CH2,), jnp.float32),
        pltpu.VMEM((_CH2,), jnp.float32),
        pltpu.SemaphoreType.DMA,
    ],
)
def _lhat_kernel(perm_hbm, rows_hbm, cols_hbm, ew_hbm, dis_hbm, out,
                 pbuf, rbuf, cbuf, wbuf, drbuf, dcbuf, obuf, sem):
    cid = lax.axis_index("c")
    sid = lax.axis_index("s")
    wid = sid * _NC + cid
    base = wid * _EPW
    for k in range(_EPW // _CH2):
        s = base + k * _CH2
        pltpu.sync_copy(perm_hbm.at[pl.ds(s, _CH2)], pbuf)
        pltpu.sync_copy(rows_hbm.at[pl.ds(s, _CH2)], rbuf)
        pltpu.sync_copy(cols_hbm.at[pl.ds(s, _CH2)], cbuf)
        pltpu.async_copy(ew_hbm.at[pbuf], wbuf, sem).wait()
        pltpu.async_copy(dis_hbm.at[rbuf], drbuf, sem).wait()
        pltpu.async_copy(dis_hbm.at[cbuf], dcbuf, sem).wait()

        def vb(j, c):
            sl = pl.ds(j * 16, 16)
            obuf[sl] = -(drbuf[sl] * wbuf[sl] * dcbuf[sl])
            return c
        lax.fori_loop(0, _CH2 // 16, vb, 0)
        pltpu.sync_copy(obuf, out.at[pl.ds(s, _CH2)])


def _vextract(vec_ref, p):
    base = pl.multiple_of((p >> 4) << 4, 16)
    v = vec_ref[pl.ds(base, 16)]
    iota = lax.broadcasted_iota(jnp.int32, (16,), 0)
    return jnp.sum(jnp.where(iota == (p & 15), v, 0))


@functools.cache
def _make_prop(D):
    @functools.partial(
        pl.kernel, mesh=_mesh, compiler_params=_SC_PARAMS,
        out_type=jax.ShapeDtypeStruct((_NPAD, D), jnp.float32),
        scratch_types=(
            [pltpu.VMEM((_PR, 16), jnp.float32)] * (D // 16) + [
                pltpu.VMEM((_CH,), jnp.int32),
                pltpu.VMEM((_CH,), jnp.int32),
                pltpu.VMEM((_CH,), jnp.float32),
                pltpu.VMEM((80,), jnp.int32),
                pltpu.VMEM((_CH, D), jnp.float32),
                pltpu.SemaphoreType.DMA,
            ]
        ),
    )
    def _prop_kernel(table, rows_hbm, crel_hbm, lhat_hbm, eo_hbm, out, *scr):
        nq = D // 16
        accs = scr[:nq]
        ibuf, cbuf, lbuf, eov, gbuf, sem = scr[nq:]
        cid = lax.axis_index("c")
        sid = lax.axis_index("s")
        wid = sid * _NC + cid
        pltpu.sync_copy(eo_hbm, eov)
        zero = jnp.zeros((16,), jnp.float32)
        iota = lax.broadcasted_iota(jnp.int32, (16,), 0)
        for sub in range(2):
            p = wid * 2 + sub

            def zb(j, c):
                for q in range(nq):
                    accs[q][j, pl.ds(0, 16)] = zero
                return c
            lax.fori_loop(0, _PR, zb, 0)

            e0 = _vextract(eov, p)
            e1 = _vextract(eov, p + 1)
            ab = (e0 >> 3) << 3
            nch = (e1 - ab + _CH - 1) >> _CHSH

            def chunk(k, c):
                ck = pl.multiple_of(ab + k * _CH, 8)
                pltpu.sync_copy(rows_hbm.at[pl.ds(ck, _CH)], ibuf)
                pltpu.sync_copy(crel_hbm.at[pl.ds(ck, _CH)], cbuf)
                pltpu.sync_copy(lhat_hbm.at[pl.ds(ck, _CH)], lbuf)
                pltpu.async_copy(table.at[ibuf], gbuf, sem).wait()
                lo = e0 - ck
                hi = e1 - ck

                def eb(g, cc):
                    b = g * 16
                    lane = b + iota
                    m = (lane >= lo) & (lane < hi)
                    crv = cbuf[pl.ds(b, 16)]
                    wv = lbuf[pl.ds(b, 16)]
                    rowi = lane
                    for l in range(16):
                        li = jnp.full((16,), l, jnp.int32)
                        for q in range(nq):
                            gl = jnp.full((16,), q * 16 + l, jnp.int32)
                            colv = plsc.load_gather(gbuf, [rowi, gl])
                            plsc.addupdate_scatter(accs[q], [crv, li],
                                                   colv * wv, mask=m)
                    return cc
                lax.fori_loop(0, _CH // 16, eb, 0)
                return c
            lax.fori_loop(0, nch, chunk, 0)
            ro = pl.ds(pl.multiple_of(p * _PR, 8), _PR)
            for q in range(nq):
                pltpu.sync_copy(accs[q], out.at[ro, pl.ds(q * 16, 16)])

    return _prop_kernel


def _dis_tc(deg2):
    def body(a_ref, o_ref):
        d = a_ref[0] + a_ref[1]
        o_ref[...] = jnp.where(d > 0, lax.rsqrt(jnp.where(d > 0, d, 1.0)), 0.0)

    out = pl.pallas_call(
        body,
        out_shape=jax.ShapeDtypeStruct((_NPAD // 128, 128), jnp.float32),
    )(deg2.reshape(2, _NPAD // 128, 128))
    return out.reshape(_NPAD)


def _gates_tc(X, Px, H, Ph, W0x, W1x, bx, W0h, W1h, bh, W0hh, bhh):
    def body(x_ref, px_ref, h_ref, ph_ref, w0x_ref, w1x_ref, bx_ref,
             w0h_ref, w1h_ref, bh_ref, w0hh_ref, bhh_ref,
             z_ref, hr_ref, s_ref):
        gx = (x_ref[...] @ w0x_ref[...] + px_ref[...] @ w1x_ref[...]
              + bx_ref[...])
        gh = (h_ref[...] @ w0h_ref[...] + ph_ref[...] @ w1h_ref[...]
              + bh_ref[...])
        Z = jax.nn.sigmoid(gx[:, :_H] + gh[:, :_H])
        R = jax.nn.sigmoid(gx[:, _H:2 * _H] + gh[:, _H:])
        HR = h_ref[...] * R
        z_ref[...] = Z
        hr_ref[...] = HR
        s_ref[...] = gx[:, 2 * _H:] + HR @ w0hh_ref[...] + bhh_ref[...]

    dx = X.shape[1]
    grid = (_N // _BLK,)
    rows = lambda d: pl.BlockSpec((_BLK, d), lambda i: (i, 0))
    whole = lambda a: pl.BlockSpec(a.shape, lambda i: (0,) * a.ndim)
    o64 = jax.ShapeDtypeStruct((_N, _H), jnp.float32)
    return pl.pallas_call(
        body,
        grid=grid,
        in_specs=[rows(dx), rows(dx), rows(_H), rows(_H),
                  whole(W0x), whole(W1x), whole(bx),
                  whole(W0h), whole(W1h), whole(bh),
                  whole(W0hh), whole(bhh)],
        out_specs=[rows(_H), rows(_H), rows(_H)],
        out_shape=[o64, o64, o64],
    )(X, Px, H, Ph, W0x, W1x, bx, W0h, W1h, bh, W0hh, bhh)


def _update_tc(Z, H, S, Phr, W1hh, LW, Lb):
    def body(z_ref, h_ref, s_ref, phr_ref, w1hh_ref, lw_ref, lb_ref,
             hn_ref, o_ref):
        Ht = jnp.tanh(s_ref[...] + phr_ref[...] @ w1hh_ref[...])
        Hn = z_ref[...] * h_ref[...] + (1.0 - z_ref[...]) * Ht
        hn_ref[...] = Hn
        o_ref[...] = jnp.maximum(Hn, 0.0) @ lw_ref[...] + lb_ref[...]

    dL = LW.shape[1]
    grid = (_N // _BLK,)
    rows = lambda d: pl.BlockSpec((_BLK, d), lambda i: (i, 0))
    whole = lambda a: pl.BlockSpec(a.shape, lambda i: (0,) * a.ndim)
    return pl.pallas_call(
        body,
        grid=grid,
        in_specs=[rows(_H), rows(_H), rows(_H), rows(_H),
                  whole(W1hh), whole(LW), whole(Lb)],
        out_specs=[rows(_H), rows(dL)],
        out_shape=[jax.ShapeDtypeStruct((_N, _H), jnp.float32),
                   jax.ShapeDtypeStruct((_N, dL), jnp.float32)],
    )(Z, H, S, Phr, W1hh, LW, Lb)


def _cat(p, pre, names, key, axis=1):
    return jnp.concatenate([p[pre + nm + key] for nm in names], axis=axis)


def kernel(x_seq, edge_index, edge_weight, h_enc, h_dec, params):
    row, col = edge_index[0], edge_index[1]

    perm = jnp.argsort(col).astype(jnp.int32)
    part = col[perm] // _PR
    key = part * 64 + (jnp.arange(_E, dtype=jnp.int32) % 64)
    perm2 = jnp.argsort(key, stable=True).astype(jnp.int32)
    perm = perm[perm2]
    part_s = part[perm2]
    row_s = row[perm]
    col_s = col[perm]
    eo = jnp.searchsorted(
        part_s, jnp.arange(_NPART + 1, dtype=jnp.int32)).astype(jnp.int32)
    eo_pad = jnp.concatenate([eo, jnp.zeros((80 - _NPART - 1,), jnp.int32)])
    crel = (col_s % _PR).astype(jnp.int32)
    zpad = jnp.zeros((_EPAD - _E,), jnp.int32)
    row_sp = jnp.concatenate([row_s, zpad])
    crel_p = jnp.concatenate([crel, zpad])

    deg2 = _deg_kernel(row, edge_weight)
    dis = _dis_tc(deg2)
    lhat_p = _lhat_kernel(perm, row_s, col_s, edge_weight, dis)

    def prop(table):
        D = table.shape[1]
        out = _make_prop(D)(table, row_sp, crel_p, lhat_p, eo_pad)
        return out[:_N]

    p = params
    xp_seq = jnp.zeros((_T, _N, 16), jnp.float32).at[:, :, :4].set(x_seq)
    zrh = ['xz', 'xr', 'xh']
    hzr = ['hz', 'hr']
    wx_e0 = jnp.zeros((16, 192), jnp.float32).at[:4].set(_cat(p, 'enc_', zrh, '_W0'))
    wx_e1 = jnp.zeros((16, 192), jnp.float32).at[:4].set(_cat(p, 'enc_', zrh, '_W1'))
    bx_e = _cat(p, 'enc_', zrh, '_b', axis=0).reshape(1, 192)
    wh_e0 = _cat(p, 'enc_', hzr, '_W0')
    wh_e1 = _cat(p, 'enc_', hzr, '_W1')
    bh_e = _cat(p, 'enc_', hzr, '_b', axis=0).reshape(1, 128)
    wx_d0 = _cat(p, 'dec_', zrh, '_W0')
    wx_d1 = _cat(p, 'dec_', zrh, '_W1')
    bx_d = _cat(p, 'dec_', zrh, '_b', axis=0).reshape(1, 192)
    wh_d0 = _cat(p, 'dec_', hzr, '_W0')
    wh_d1 = _cat(p, 'dec_', hzr, '_W1')
    bh_d = _cat(p, 'dec_', hzr, '_b', axis=0).reshape(1, 128)
    bhh_e = p['enc_hh_b'].reshape(1, _H)
    bhh_d = p['dec_hh_b'].reshape(1, _H)
    elb = p['enc_lin_b'].reshape(1, -1)
    dlb = p['dec_lin_b'].reshape(1, -1)

    y = None
    for t in range(_T):
        xp = xp_seq[t]
        Px = prop(xp)
        Ph = prop(h_enc)
        Z, HR, S = _gates_tc(xp, Px, h_enc, Ph, wx_e0, wx_e1, bx_e,
                             wh_e0, wh_e1, bh_e, p['enc_hh_W0'], bhh_e)
        Phr = prop(HR)
        h_enc, z = _update_tc(Z, h_enc, S, Phr, p['enc_hh_W1'],
                              p['enc_lin_W'], elb)
        Pz = prop(z)
        Phd = prop(h_dec)
        Z2, HR2, S2 = _gates_tc(z, Pz, h_dec, Phd, wx_d0, wx_d1, bx_d,
                                wh_d0, wh_d1, bh_d, p['dec_hh_W0'], bhh_d)
        Phdr = prop(HR2)
        h_dec, y = _update_tc(Z2, h_dec, S2, Phdr, p['dec_hh_W1'],
                              p['dec_lin_W'], dlb)
    return y
